# baseline (speedup 1.0000x reference)
import jax
import jax.numpy as jnp
from jax.experimental import pallas as pl
from jax.experimental.pallas import tpu as pltpu

B, D, H, K0, K1 = 1024, 128, 256, 1000, 100000
K0P = 1024
TK = 512
NT = (K1 + TK - 1) // TK

_PREC = jax.lax.Precision.HIGHEST
_LOG2E = 1.4426950408889634
_LN2 = 0.6931471805599453


def _body(states_ref, l1b_ref, W1_ref, b1_ref, Wl0_ref, E0_ref, wb_ref,
          l0c_ref, l1c_ref, Wl1_ref, lp_ref, ent_ref,
          h1_s, sacc, tacc, cacc, plp_s, pent_s, ybuf):
    k = pl.program_id(0)

    @pl.when(k == 0)
    def _small_stages():
        trunk = jnp.tanh(
            jnp.dot(states_ref[...], W1_ref[...],
                    preferred_element_type=jnp.float32, precision=_PREC)
            + b1_ref[...])
        l0_cat = l0c_ref[...]
        cols0 = jax.lax.broadcasted_iota(jnp.int32, (B, K0P), 1)
        l0 = jnp.dot(trunk.astype(jnp.bfloat16),
                     Wl0_ref[...].astype(jnp.bfloat16),
                     preferred_element_type=jnp.float32)
        l0 = jnp.where(cols0 < K0, l0, -1e30)
        e0 = jnp.exp(l0)
        s0 = jnp.sum(e0, axis=1, keepdims=True)
        t0 = jnp.sum(e0 * l0, axis=1, keepdims=True)
        c0 = jnp.sum(jnp.where(cols0 == l0_cat, l0, 0.0), axis=1, keepdims=True)
        logz0 = jnp.log(s0)
        lp0 = c0 - logz0
        ent0 = logz0 - t0 / s0
        onehot = (cols0 == l0_cat).astype(jnp.bfloat16)
        h1 = jnp.tanh(trunk + jnp.dot(onehot, E0_ref[...].astype(jnp.bfloat16),
                                      preferred_element_type=jnp.float32))
        h1_s[...] = (h1 * _LOG2E).astype(jnp.bfloat16)
        bl = jnp.dot(h1, wb_ref[...],
                     preferred_element_type=jnp.float32, precision=_PREC)
        ax = jnp.abs(bl)
        lse = jnp.log(1.0 + jnp.exp(-ax))
        lsp = jnp.where(bl >= 0, -lse, bl - lse)
        lsn = lsp - bl
        pb = jnp.exp(lsp)
        lb = l1b_ref[...]
        lpb = lb * lsp + (1.0 - lb) * lsn
        entb = -(pb * lsp + (1.0 - pb) * lsn)
        plp_s[...] = lp0 + lpb
        pent_s[...] = ent0 + entb
        sacc[...] = jnp.zeros((B, 128), jnp.float32)
        tacc[...] = jnp.zeros((B, 128), jnp.float32)
        cacc[...] = jnp.zeros((B, 128), jnp.float32)

    @pl.when(k < NT)
    def _mm():
        ybuf[k % 2] = jnp.dot(h1_s[...], Wl1_ref[...].astype(jnp.bfloat16),
                              preferred_element_type=jnp.float32)

    cols = jax.lax.broadcasted_iota(jnp.int32, (B, TK), 1)
    rel = l1c_ref[...] - (k - 1) * TK

    def fold(x):
        acc = x[:, 0:128]
        for i in range(1, TK // 128):
            acc = acc + x[:, i * 128:(i + 1) * 128]
        return acc

    @pl.when(jnp.logical_and(k >= 1, k < NT))
    def _acc():
        y = ybuf[(k - 1) % 2]
        e = jnp.exp2(y)
        sacc[...] += fold(e)
        tacc[...] += fold(e * y)
        cacc[...] += fold(jnp.where(cols == rel, y, 0.0))

    @pl.when(k == NT)
    def _finish():
        y = ybuf[(NT - 1) % 2]
        ym = jnp.where(cols < K1 - (NT - 1) * TK, y, -1e30)
        e = jnp.exp2(ym)
        sacc[...] += fold(e)
        tacc[...] += fold(e * ym)
        cacc[...] += fold(jnp.where(cols == rel, y, 0.0))
        s1 = jnp.sum(sacc[...], axis=1, keepdims=True)
        t1 = jnp.sum(tacc[...], axis=1, keepdims=True) * _LN2
        c1 = jnp.sum(cacc[...], axis=1, keepdims=True) * _LN2
        logz1 = jnp.log(s1)
        lp1 = c1 - logz1
        ent1 = logz1 - t1 / s1
        lp_ref[...] = plp_s[...] + lp1
        ent_ref[...] = pent_s[...] + ent1


def kernel(states, l1_bern, W1, b1, W_l0, E0, W_l1, w_bern, l0_cat, l1_cat):
    Wl0p = jnp.pad(W_l0, ((0, 0), (0, K0P - K0)))
    E0p = jnp.pad(E0, ((0, K0P - K0), (0, 0)))
    b1r = b1.reshape(1, H)
    wbr = w_bern.reshape(H, 1)
    l0c = l0_cat.reshape(B, 1).astype(jnp.int32)
    l1c = l1_cat.reshape(B, 1).astype(jnp.int32)
    l1b = l1_bern.reshape(B, 1)

    fixed = lambda k: (0, 0)
    out = pl.pallas_call(
        _body,
        grid=(NT + 1,),
        in_specs=[
            pl.BlockSpec((B, D), fixed),
            pl.BlockSpec((B, 1), fixed),
            pl.BlockSpec((D, H), fixed),
            pl.BlockSpec((1, H), fixed),
            pl.BlockSpec((H, K0P), fixed),
            pl.BlockSpec((K0P, H), fixed),
            pl.BlockSpec((H, 1), fixed),
            pl.BlockSpec((B, 1), fixed),
            pl.BlockSpec((B, 1), fixed),
            pl.BlockSpec((H, TK),
                         lambda k: (0, jnp.minimum(k, NT - 1))),
        ],
        out_specs=[pl.BlockSpec((B, 1), fixed), pl.BlockSpec((B, 1), fixed)],
        out_shape=[jax.ShapeDtypeStruct((B, 1), jnp.float32),
                   jax.ShapeDtypeStruct((B, 1), jnp.float32)],
        scratch_shapes=[
            pltpu.VMEM((B, H), jnp.bfloat16),
            pltpu.VMEM((B, 128), jnp.float32),
            pltpu.VMEM((B, 128), jnp.float32),
            pltpu.VMEM((B, 128), jnp.float32),
            pltpu.VMEM((B, 1), jnp.float32),
            pltpu.VMEM((B, 1), jnp.float32),
            pltpu.VMEM((2, B, TK), jnp.float32),
        ],
    )(states, l1b, W1, b1r, Wl0p, E0p, wbr, l0c, l1c, W_l1)
    lp, ent = out
    return lp, ent


# speedup vs baseline: 1.1700x; 1.1700x over previous
import jax
import jax.numpy as jnp
from jax.experimental import pallas as pl
from jax.experimental.pallas import tpu as pltpu

B, D, H, K0, K1 = 1024, 128, 256, 1000, 100000
K0P = 1024
TK = 1024
NT = (K1 + TK - 1) // TK

_PREC = jax.lax.Precision.HIGHEST
_LOG2E = 1.4426950408889634
_LN2 = 0.6931471805599453


def _body(states_ref, l1b_ref, W1_ref, b1_ref, Wl0_ref, E0_ref, wb_ref,
          l0c_ref, l1c_ref, Wl1_ref, lp_ref, ent_ref,
          h1_s, sacc, tacc, cacc, plp_s, pent_s, ybuf):
    k = pl.program_id(0)

    @pl.when(k == 0)
    def _small_stages():
        trunk = jnp.tanh(
            jnp.dot(states_ref[...], W1_ref[...],
                    preferred_element_type=jnp.float32, precision=_PREC)
            + b1_ref[...])
        l0_cat = l0c_ref[...]
        cols0 = jax.lax.broadcasted_iota(jnp.int32, (B, K0P), 1)
        l0 = jnp.dot(trunk.astype(jnp.bfloat16),
                     Wl0_ref[...].astype(jnp.bfloat16),
                     preferred_element_type=jnp.float32)
        l0 = jnp.where(cols0 < K0, l0, -1e30)
        e0 = jnp.exp(l0)
        s0 = jnp.sum(e0, axis=1, keepdims=True)
        t0 = jnp.sum(e0 * l0, axis=1, keepdims=True)
        c0 = jnp.sum(jnp.where(cols0 == l0_cat, l0, 0.0), axis=1, keepdims=True)
        logz0 = jnp.log(s0)
        lp0 = c0 - logz0
        ent0 = logz0 - t0 / s0
        onehot = (cols0 == l0_cat).astype(jnp.bfloat16)
        h1 = jnp.tanh(trunk + jnp.dot(onehot, E0_ref[...].astype(jnp.bfloat16),
                                      preferred_element_type=jnp.float32))
        h1_s[...] = (h1 * _LOG2E).astype(jnp.bfloat16)
        bl = jnp.dot(h1, wb_ref[...],
                     preferred_element_type=jnp.float32, precision=_PREC)
        ax = jnp.abs(bl)
        lse = jnp.log(1.0 + jnp.exp(-ax))
        lsp = jnp.where(bl >= 0, -lse, bl - lse)
        lsn = lsp - bl
        pb = jnp.exp(lsp)
        lb = l1b_ref[...]
        lpb = lb * lsp + (1.0 - lb) * lsn
        entb = -(pb * lsp + (1.0 - pb) * lsn)
        plp_s[...] = lp0 + lpb
        pent_s[...] = ent0 + entb
        sacc[...] = jnp.zeros((B, 128), jnp.float32)
        tacc[...] = jnp.zeros((B, 128), jnp.float32)
        cacc[...] = jnp.zeros((B, 128), jnp.float32)

    @pl.when(k < NT)
    def _mm():
        ybuf[k % 2] = jnp.dot(h1_s[...], Wl1_ref[...].astype(jnp.bfloat16),
                              preferred_element_type=jnp.float32)

    cols = jax.lax.broadcasted_iota(jnp.int32, (B, TK), 1)
    rel = l1c_ref[...] - (k - 1) * TK

    def fold(x):
        acc = x[:, 0:128]
        for i in range(1, TK // 128):
            acc = acc + x[:, i * 128:(i + 1) * 128]
        return acc

    @pl.when(jnp.logical_and(k >= 1, k < NT))
    def _acc():
        y = ybuf[(k - 1) % 2]
        e = jnp.exp2(y)
        sacc[...] += fold(e)
        tacc[...] += fold(e * y)
        cacc[...] += fold(jnp.where(cols == rel, y, 0.0))

    @pl.when(k == NT)
    def _finish():
        y = ybuf[(NT - 1) % 2]
        ym = jnp.where(cols < K1 - (NT - 1) * TK, y, -1e30)
        e = jnp.exp2(ym)
        sacc[...] += fold(e)
        tacc[...] += fold(e * ym)
        cacc[...] += fold(jnp.where(cols == rel, y, 0.0))
        s1 = jnp.sum(sacc[...], axis=1, keepdims=True)
        t1 = jnp.sum(tacc[...], axis=1, keepdims=True) * _LN2
        c1 = jnp.sum(cacc[...], axis=1, keepdims=True) * _LN2
        logz1 = jnp.log(s1)
        lp1 = c1 - logz1
        ent1 = logz1 - t1 / s1
        lp_ref[...] = plp_s[...] + lp1
        ent_ref[...] = pent_s[...] + ent1


def kernel(states, l1_bern, W1, b1, W_l0, E0, W_l1, w_bern, l0_cat, l1_cat):
    Wl0p = jnp.pad(W_l0, ((0, 0), (0, K0P - K0)))
    E0p = jnp.pad(E0, ((0, K0P - K0), (0, 0)))
    b1r = b1.reshape(1, H)
    wbr = w_bern.reshape(H, 1)
    l0c = l0_cat.reshape(B, 1).astype(jnp.int32)
    l1c = l1_cat.reshape(B, 1).astype(jnp.int32)
    l1b = l1_bern.reshape(B, 1)

    fixed = lambda k: (0, 0)
    out = pl.pallas_call(
        _body,
        grid=(NT + 1,),
        in_specs=[
            pl.BlockSpec((B, D), fixed),
            pl.BlockSpec((B, 1), fixed),
            pl.BlockSpec((D, H), fixed),
            pl.BlockSpec((1, H), fixed),
            pl.BlockSpec((H, K0P), fixed),
            pl.BlockSpec((K0P, H), fixed),
            pl.BlockSpec((H, 1), fixed),
            pl.BlockSpec((B, 1), fixed),
            pl.BlockSpec((B, 1), fixed),
            pl.BlockSpec((H, TK),
                         lambda k: (0, jnp.minimum(k, NT - 1))),
        ],
        out_specs=[pl.BlockSpec((B, 1), fixed), pl.BlockSpec((B, 1), fixed)],
        out_shape=[jax.ShapeDtypeStruct((B, 1), jnp.float32),
                   jax.ShapeDtypeStruct((B, 1), jnp.float32)],
        scratch_shapes=[
            pltpu.VMEM((B, H), jnp.bfloat16),
            pltpu.VMEM((B, 128), jnp.float32),
            pltpu.VMEM((B, 128), jnp.float32),
            pltpu.VMEM((B, 128), jnp.float32),
            pltpu.VMEM((B, 1), jnp.float32),
            pltpu.VMEM((B, 1), jnp.float32),
            pltpu.VMEM((2, B, TK), jnp.float32),
        ],
    )(states, l1b, W1, b1r, Wl0p, E0p, wbr, l0c, l1c, W_l1)
    lp, ent = out
    return lp, ent
